# baseline (speedup 1.0000x reference)
import functools

import jax
import jax.numpy as jnp
from jax import lax
from jax.experimental import pallas as pl
from jax.experimental.pallas import tpu as pltpu
from jax.experimental.pallas import tpu_sc as plsc

NUM_CORES = 2
NUM_SUBCORES = 16
NW = NUM_CORES * NUM_SUBCORES

ROWS = 128
SLAB = 40
NBUF = 4
LANES = 16


def _build_kernel(n_rows, d_model, n_blocks):
    mesh = plsc.VectorSubcoreMesh(core_axis_name="c", subcore_axis_name="s")
    n_slabs = n_blocks // SLAB

    @functools.partial(
        pl.kernel,
        out_type=jax.ShapeDtypeStruct((NW, n_blocks, ROWS, d_model), jnp.float32),
        mesh=mesh,
        scratch_types=[
            pltpu.VMEM_SHARED((n_rows, d_model), jnp.float32),
            pltpu.VMEM((n_rows, d_model), jnp.float32),
            [pltpu.VMEM((SLAB, ROWS), jnp.int32) for _ in range(2)],
            [pltpu.VMEM((ROWS, d_model), jnp.float32) for _ in range(NBUF)],
            [pltpu.SemaphoreType.DMA for _ in range(NBUF)],
            [pltpu.SemaphoreType.DMA for _ in range(NBUF)],
            pltpu.SemaphoreType.DMA,
        ],
    )
    def gather_kernel(idx_hbm, pe_hbm, out_hbm, table_sp, table_l, idx_bufs,
                      rows, gsems, wsems, isem):
        c = lax.axis_index("c")
        s = lax.axis_index("s")
        wid = c * NUM_SUBCORES + s

        @pl.when(s == 0)
        def _():
            pltpu.sync_copy(pe_hbm, table_sp)
        pltpu.sync_copy(pe_hbm, table_l)
        plsc.subcore_barrier()

        def compute_block(idx_v, jj, buf):
            def grp_body(g, _):
                v16 = idx_v[jj, pl.ds(LANES * g, LANES)]
                for li in range(LANES):
                    ri = v16[li]
                    row = LANES * g + li
                    for k in range(d_model // LANES):
                        buf[row, pl.ds(LANES * k, LANES)] = (
                            table_l[ri, pl.ds(LANES * k, LANES)])
                return 0

            lax.fori_loop(0, ROWS // LANES, grp_body, 0)

        def quad_body(si, idx_v, q):
            base = si * SLAB + NBUF * q
            for b in range(NBUF - 1):
                j = base + b

                @pl.when(j >= NBUF)
                def _():
                    pltpu.make_async_copy(
                        rows[b], out_hbm.at[wid, j], wsems[b]
                    ).wait()

                pltpu.async_copy(
                    table_sp.at[idx_v.at[NBUF * q + b]], rows[b], gsems[b]
                )

                if b > 0:
                    pb = b - 1
                    pj = j - 1
                else:
                    pb = NBUF - 2
                    pj = j - 2

                def flush_prev():
                    pltpu.make_async_copy(
                        table_sp.at[idx_v.at[0]], rows[pb], gsems[pb]
                    ).wait()
                    pltpu.async_copy(rows[pb], out_hbm.at[wid, pj], wsems[pb])

                if b > 0:
                    flush_prev()
                else:
                    pl.when(q >= 1)(flush_prev)

            bc = NBUF - 1
            jc = base + bc

            @pl.when(jc >= NBUF)
            def _():
                pltpu.make_async_copy(
                    rows[bc], out_hbm.at[wid, jc], wsems[bc]
                ).wait()

            compute_block(idx_v, NBUF * q + bc, rows[bc])
            pltpu.async_copy(rows[bc], out_hbm.at[wid, jc], wsems[bc])

        def slab_flush(si, idx_v):
            last = NBUF - 2
            pltpu.make_async_copy(
                table_sp.at[idx_v.at[0]], rows[last], gsems[last]
            ).wait()
            pltpu.async_copy(
                rows[last], out_hbm.at[wid, si * SLAB + SLAB - 2], wsems[last]
            )

        def slab_pair_body(sp, _):
            si0 = 2 * sp
            si1 = 2 * sp + 1

            pltpu.async_copy(
                idx_hbm.at[wid, pl.ds(si1 * SLAB, SLAB)], idx_bufs[1], isem
            )
            lax.fori_loop(0, SLAB // NBUF,
                          lambda q, _: (quad_body(si0, idx_bufs[0], q), 0)[1], 0)
            slab_flush(si0, idx_bufs[0])
            pltpu.make_async_copy(
                idx_hbm.at[wid, pl.ds(si1 * SLAB, SLAB)], idx_bufs[1], isem
            ).wait()

            @pl.when(sp + 1 < n_slabs // 2)
            def _():
                pltpu.async_copy(
                    idx_hbm.at[wid, pl.ds((si0 + 2) * SLAB, SLAB)],
                    idx_bufs[0], isem,
                )
            lax.fori_loop(0, SLAB // NBUF,
                          lambda q, _: (quad_body(si1, idx_bufs[1], q), 0)[1], 0)
            slab_flush(si1, idx_bufs[1])

            @pl.when(sp + 1 < n_slabs // 2)
            def _():
                pltpu.make_async_copy(
                    idx_hbm.at[wid, pl.ds((si0 + 2) * SLAB, SLAB)],
                    idx_bufs[0], isem,
                ).wait()
            return 0

        pltpu.sync_copy(idx_hbm.at[wid, pl.ds(0, SLAB)], idx_bufs[0])
        lax.fori_loop(0, n_slabs // 2, slab_pair_body, 0)

        for b in range(NBUF):
            pltpu.make_async_copy(
                rows[b], out_hbm.at[wid, n_blocks - 1], wsems[b]
            ).wait()

    return gather_kernel


def kernel(doy, pe):
    batch, hist = doy.shape
    n_rows, d_model = pe.shape
    total = batch * hist
    assert total % (NW * ROWS) == 0
    n_blocks = total // (NW * ROWS)
    assert n_blocks % (2 * SLAB) == 0 and SLAB % NBUF == 0

    idx = doy.reshape(NW, n_blocks, ROWS).astype(jnp.int32)
    out = _build_kernel(n_rows, d_model, n_blocks)(idx, pe)
    return out.reshape(batch, hist, d_model)
